# baseline (speedup 1.0000x reference)
import functools

import jax
import jax.numpy as jnp
from jax import lax
from jax.experimental import pallas as pl
from jax.experimental.pallas import tpu as pltpu
from jax.experimental.pallas import tpu_sc as plsc

N = 10000
E = 320000
D = 128
HALF = D // 2
L = 3

NC = 2
NS = 16
CHUNK = 128
NSLOT = 4
N_PAD = 10240
ROWS_PER_TILE = N_PAD // NS
CPT = 160
E_PAD = NS * CPT * CHUNK
NROUND = CPT // NSLOT
IDX_BITS = 14
IDX_MASK = (1 << IDX_BITS) - 1


def _sc_aggregate_body(hst_hbm, edges_hbm, sums_hbm, pk_v, zero_v,
                       idxg_vs, idxs_vs, rows_vs, acc_sh, h_sh, semp, semgs):
  c = lax.axis_index("c")
  s = lax.axis_index("s")

  def zfill_body(i, carry):
    for j in range(HALF // 16):
      zero_v[i, pl.ds(j * 16, 16)] = jnp.zeros((16,), jnp.float32)
    return carry

  lax.fori_loop(0, CHUNK, zfill_body, 0)
  base = s * ROWS_PER_TILE
  ebase = s * (CPT * CHUNK)

  def load_grp(r):
    pltpu.async_copy(
        edges_hbm.at[pl.ds(ebase + r * NSLOT * CHUNK, NSLOT * CHUNK)],
        pk_v, semp)

  def drain_grp():
    pltpu.make_async_copy(
        edges_hbm.at[pl.ds(0, NSLOT * CHUNK)], pk_v, semp).wait()

  def unpack_slot(t):
    for k in range(CHUNK // 16):
      p = pk_v[pl.ds(t * CHUNK + k * 16, 16)]
      lo = p & IDX_MASK
      hi = lax.shift_right_logical(p, IDX_BITS)
      idxg_vs[t][pl.ds(k * 16, 16)] = jnp.where(c == 0, lo, hi)
      idxs_vs[t][pl.ds(k * 16, 16)] = jnp.where(c == 0, hi, lo)

  def fire(t):
    pltpu.async_copy(h_sh.at[idxg_vs[t]], rows_vs[t], semgs[t])

  def drain_g(t):
    pltpu.make_async_copy(
        sums_hbm.at[pl.ds(0, CHUNK)], rows_vs[t], semgs[t]).wait()

  def scatter(t):
    pltpu.sync_copy(rows_vs[t], acc_sh.at[idxs_vs[t]], add=True)

  def phase_body(p, carry):
    for k in range(ROWS_PER_TILE // CHUNK):
      pltpu.sync_copy(zero_v, acc_sh.at[pl.ds(base + k * CHUNK, CHUNK)])
    pltpu.sync_copy(hst_hbm.at[pl.ds(p * N_PAD + base, ROWS_PER_TILE)],
                    h_sh.at[pl.ds(base, ROWS_PER_TILE)])
    plsc.subcore_barrier()

    load_grp(0)
    drain_grp()
    for t in range(NSLOT):
      unpack_slot(t)
      fire(t)
    load_grp(1)

    def round_body(r, carry2):
      drain_g(0)
      scatter(0)
      drain_grp()
      unpack_slot(0)
      fire(0)
      for t in range(1, NSLOT):
        drain_g(t)
        scatter(t)
        unpack_slot(t)
        fire(t)

      @pl.when(r < NROUND - 2)
      def _():
        load_grp(r + 2)
      return carry2

    lax.fori_loop(0, NROUND - 1, round_body, 0)
    for t in range(NSLOT):
      drain_g(t)
      scatter(t)
    plsc.subcore_barrier()

    out0 = (c * 2 + p) * N_PAD + base
    pltpu.sync_copy(acc_sh.at[pl.ds(base, ROWS_PER_TILE)],
                    sums_hbm.at[pl.ds(out0, ROWS_PER_TILE)])
    plsc.subcore_barrier()
    return carry

  lax.fori_loop(0, 2, phase_body, 0)


_sc_aggregate = pl.kernel(
    _sc_aggregate_body,
    out_type=jax.ShapeDtypeStruct((4 * N_PAD, HALF), jnp.float32),
    mesh=plsc.VectorSubcoreMesh(core_axis_name="c", subcore_axis_name="s"),
    compiler_params=pltpu.CompilerParams(use_tc_tiling_on_sc=False),
    scratch_types=[
        pltpu.VMEM((NSLOT * CHUNK,), jnp.int32),
        pltpu.VMEM((CHUNK, HALF), jnp.float32),
        [pltpu.VMEM((CHUNK,), jnp.int32) for _ in range(NSLOT)],
        [pltpu.VMEM((CHUNK,), jnp.int32) for _ in range(NSLOT)],
        [pltpu.VMEM((CHUNK, HALF), jnp.float32) for _ in range(NSLOT)],
        pltpu.VMEM_SHARED((N_PAD, HALF), jnp.float32),
        pltpu.VMEM_SHARED((N_PAD, HALF), jnp.float32),
        pltpu.SemaphoreType.DMA,
        [pltpu.SemaphoreType.DMA for _ in range(NSLOT)],
    ],
)


def _sc_counts_body(edges_hbm, cnts_hbm, pk0_v, pk1_v, idxg_v, idxs0_v,
                    idxs1_v, ones_v, zeros_v, cnt_sh, semp0, semp1):
  c = lax.axis_index("c")
  s = lax.axis_index("s")

  for j in range(128 // 16):
    zeros_v[pl.ds(j * 16, 16)] = jnp.zeros((16,), jnp.float32)
  for j in range(CHUNK // 16):
    ones_v[pl.ds(j * 16, 16)] = jnp.ones((16,), jnp.float32)

  base = s * ROWS_PER_TILE
  for k in range(ROWS_PER_TILE // 128):
    pltpu.sync_copy(zeros_v, cnt_sh.at[pl.ds(base + k * 128, 128)])
  plsc.subcore_barrier()

  ebase = s * (CPT * CHUNK)

  def load_pk(jj, pk_buf, sem):
    pltpu.async_copy(edges_hbm.at[pl.ds(ebase + jj * CHUNK, CHUNK)],
                     pk_buf, sem)

  def drain_pk(pk_buf, sem):
    pltpu.make_async_copy(edges_hbm.at[pl.ds(0, CHUNK)], pk_buf, sem).wait()

  def unpack(pk_v, idxs_v):
    for k in range(CHUNK // 16):
      p = pk_v[pl.ds(k * 16, 16)]
      lo = p & IDX_MASK
      hi = lax.shift_right_logical(p, IDX_BITS)
      idxs_v[pl.ds(k * 16, 16)] = jnp.where(c == 0, hi, lo)

  load_pk(0, pk0_v, semp0)

  def body(g, carry):
    j0 = 2 * g
    load_pk(j0 + 1, pk1_v, semp1)
    drain_pk(pk0_v, semp0)
    unpack(pk0_v, idxs0_v)

    @pl.when(g < CPT // 2 - 1)
    def _():
      load_pk(j0 + 2, pk0_v, semp0)

    pltpu.sync_copy(ones_v, cnt_sh.at[idxs0_v], add=True)
    drain_pk(pk1_v, semp1)
    unpack(pk1_v, idxs1_v)
    pltpu.sync_copy(ones_v, cnt_sh.at[idxs1_v], add=True)
    return carry

  lax.fori_loop(0, CPT // 2, body, 0)
  plsc.subcore_barrier()

  @pl.when(c == 0)
  def _():
    pltpu.sync_copy(cnt_sh.at[pl.ds(base, ROWS_PER_TILE)],
                    cnts_hbm.at[pl.ds(base, ROWS_PER_TILE)])

  @pl.when(c == 1)
  def _():
    pltpu.sync_copy(cnt_sh.at[pl.ds(base, ROWS_PER_TILE)],
                    cnts_hbm.at[pl.ds(N_PAD + base, ROWS_PER_TILE)])


_sc_counts = pl.kernel(
    _sc_counts_body,
    out_type=jax.ShapeDtypeStruct((2 * N_PAD,), jnp.float32),
    mesh=plsc.VectorSubcoreMesh(core_axis_name="c", subcore_axis_name="s"),
    scratch_types=[
        pltpu.VMEM((CHUNK,), jnp.int32),
        pltpu.VMEM((CHUNK,), jnp.int32),
        pltpu.VMEM((CHUNK,), jnp.int32),
        pltpu.VMEM((CHUNK,), jnp.int32),
        pltpu.VMEM((CHUNK,), jnp.int32),
        pltpu.VMEM((CHUNK,), jnp.float32),
        pltpu.VMEM((128,), jnp.float32),
        pltpu.VMEM_SHARED((N_PAD,), jnp.float32),
        pltpu.SemaphoreType.DMA,
        pltpu.SemaphoreType.DMA,
    ],
)


def _tc_layer_body(hlo_ref, hhi_ref, s00_ref, s01_ref, s10_ref, s11_ref,
                   c1_ref, c2_ref, w1t_ref, w2t_ref, wrt_ref, br_ref,
                   o_ref, *, blk, nb):
  r1 = 1.0 / jnp.maximum(c1_ref[...], 1.0)
  r2 = 1.0 / jnp.maximum(c2_ref[...], 1.0)
  h = jnp.concatenate([hlo_ref[...], hhi_ref[...]], axis=1)
  m1 = jnp.concatenate([s00_ref[...] * r1, s01_ref[...] * r1], axis=1)
  m2 = jnp.concatenate([s10_ref[...] * r2, s11_ref[...] * r2], axis=1)
  acc = jnp.dot(h, wrt_ref[...], preferred_element_type=jnp.float32)
  acc = acc + jnp.dot(m1, w1t_ref[...], preferred_element_type=jnp.float32)
  acc = acc + jnp.dot(m2, w2t_ref[...], preferred_element_type=jnp.float32)
  acc = acc + br_ref[...]
  acc = jnp.maximum(acc, 0.0)
  i = pl.program_id(0)
  rows = lax.broadcasted_iota(jnp.int32, acc.shape, 0) + (i % nb) * blk
  acc = jnp.where(rows < N, acc, 0.0)
  o_ref[...] = jnp.where(i < nb, acc[:, :HALF], acc[:, HALF:])


def _tc_layer(hst, sums_flat, cnts_flat, w1t, w2t, wrt, brl):
  blk = 1024
  nb = N_PAD // blk
  lo = lambda k: pl.BlockSpec((blk, HALF), lambda i, k=k: (i % nb + k * nb, 0))
  cspec = lambda k: pl.BlockSpec((blk, 1), lambda i, k=k: (i % nb + k * nb, 0))
  full_spec = lambda a, b: pl.BlockSpec((a, b), lambda i: (0, 0))
  return pl.pallas_call(
      functools.partial(_tc_layer_body, blk=blk, nb=nb),
      grid=(2 * nb,),
      in_specs=[
          lo(0), lo(1),
          lo(0), lo(1), lo(2), lo(3),
          cspec(0), cspec(1),
          full_spec(D, D), full_spec(D, D), full_spec(D, D), full_spec(1, D),
      ],
      out_specs=pl.BlockSpec((blk, HALF), lambda i: (i, 0)),
      out_shape=jax.ShapeDtypeStruct((2 * N_PAD, HALF), jnp.float32),
  )(hst, hst, sums_flat, sums_flat, sums_flat, sums_flat,
    cnts_flat, cnts_flat, w1t, w2t, wrt, brl)


def _tc_final_body(x_lo, x_hi, h1_lo, h1_hi, h2_lo, h2_hi, h3_lo, h3_hi,
                   wft_ref, bf_ref, o_ref):
  cat = jnp.concatenate(
      [x_lo[...], x_hi[...], h1_lo[...], h1_hi[...],
       h2_lo[...], h2_hi[...], h3_lo[...], h3_hi[...]], axis=1)
  o_ref[...] = jnp.dot(
      cat, wft_ref[...], preferred_element_type=jnp.float32) + bf_ref[...]


def _tc_final(stacked, wft, bf):
  blk = 1024
  nb = N_PAD // blk
  lo = pl.BlockSpec((blk, HALF), lambda i: (i, 0))
  hi = pl.BlockSpec((blk, HALF), lambda i, nb=nb: (i + nb, 0))
  args = []
  specs = []
  for a in stacked:
    args.extend([a, a])
    specs.extend([lo, hi])
  return pl.pallas_call(
      _tc_final_body,
      grid=(nb,),
      in_specs=specs + [
          pl.BlockSpec((L * D + D, D), lambda i: (0, 0)),
          pl.BlockSpec((1, D), lambda i: (0, 0)),
      ],
      out_specs=pl.BlockSpec((blk, D), lambda i: (i, 0)),
      out_shape=jax.ShapeDtypeStruct((N_PAD, D), jnp.float32),
  )(*args, wft, bf)


def kernel(x, edge_index, W1, W2, Wr, br, Wf, bf):
  x_pad = jnp.zeros((N_PAD, D), jnp.float32).at[:N].set(x)
  xst = jnp.concatenate([x_pad[:, :HALF], x_pad[:, HALF:]], axis=0)
  pad = jnp.full((E_PAD - E,), N, jnp.int32)
  src_p = jnp.concatenate([edge_index[0], pad])
  dst_p = jnp.concatenate([edge_index[1], pad])
  edges_p = src_p | (dst_p << IDX_BITS)
  w1t = jnp.transpose(W1, (0, 2, 1))
  w2t = jnp.transpose(W2, (0, 2, 1))
  wrt = jnp.transpose(Wr, (0, 2, 1))
  wft = jnp.transpose(Wf)
  hst = xst
  stacked = [xst]
  cnts_flat = _sc_counts(edges_p).reshape(2 * N_PAD, 1)
  for l in range(L):
    sums_flat = _sc_aggregate(hst, edges_p)
    hst = _tc_layer(hst, sums_flat, cnts_flat,
                    w1t[l], w2t[l], wrt[l], br[l].reshape(1, D))
    stacked.append(hst)
  out = _tc_final(stacked, wft, bf.reshape(1, D))
  return out[:N]


# speedup vs baseline: 1.0005x; 1.0005x over previous
import functools

import jax
import jax.numpy as jnp
from jax import lax
from jax.experimental import pallas as pl
from jax.experimental.pallas import tpu as pltpu
from jax.experimental.pallas import tpu_sc as plsc

N = 10000
E = 320000
D = 128
HALF = D // 2
L = 3

NC = 2
NS = 16
CHUNK = 128
NSLOT = 2
N_PAD = 10240
ROWS_PER_TILE = N_PAD // NS
CPT = 158
E_PAD = NS * CPT * CHUNK
NROUND = CPT // NSLOT
IDX_BITS = 14
IDX_MASK = (1 << IDX_BITS) - 1


def _sc_aggregate_body(hst_hbm, edges_hbm, sums_hbm, pk_v, zero_v,
                       idxg_vs, idxs_vs, rows_vs, acc_sh, h_sh, semp, semgs):
  c = lax.axis_index("c")
  s = lax.axis_index("s")

  def zfill_body(i, carry):
    for j in range(HALF // 16):
      zero_v[i, pl.ds(j * 16, 16)] = jnp.zeros((16,), jnp.float32)
    return carry

  lax.fori_loop(0, CHUNK, zfill_body, 0)
  base = s * ROWS_PER_TILE
  ebase = s * (CPT * CHUNK)

  def load_grp(r):
    pltpu.async_copy(
        edges_hbm.at[pl.ds(ebase + r * NSLOT * CHUNK, NSLOT * CHUNK)],
        pk_v, semp)

  def drain_grp():
    pltpu.make_async_copy(
        edges_hbm.at[pl.ds(0, NSLOT * CHUNK)], pk_v, semp).wait()

  def unpack_slot(t):
    for k in range(CHUNK // 16):
      p = pk_v[pl.ds(t * CHUNK + k * 16, 16)]
      lo = p & IDX_MASK
      hi = lax.shift_right_logical(p, IDX_BITS)
      idxg_vs[t][pl.ds(k * 16, 16)] = jnp.where(c == 0, lo, hi)
      idxs_vs[t][pl.ds(k * 16, 16)] = jnp.where(c == 0, hi, lo)

  def fire(t):
    pltpu.async_copy(h_sh.at[idxg_vs[t]], rows_vs[t], semgs[t])

  def drain_g(t):
    pltpu.make_async_copy(
        sums_hbm.at[pl.ds(0, CHUNK)], rows_vs[t], semgs[t]).wait()

  def scatter(t):
    pltpu.sync_copy(rows_vs[t], acc_sh.at[idxs_vs[t]], add=True)

  def phase_body(p, carry):
    for k in range(ROWS_PER_TILE // CHUNK):
      pltpu.sync_copy(zero_v, acc_sh.at[pl.ds(base + k * CHUNK, CHUNK)])
    pltpu.sync_copy(hst_hbm.at[pl.ds(p * N_PAD + base, ROWS_PER_TILE)],
                    h_sh.at[pl.ds(base, ROWS_PER_TILE)])
    plsc.subcore_barrier()

    load_grp(0)
    drain_grp()
    for t in range(NSLOT):
      unpack_slot(t)
      fire(t)
    load_grp(1)

    def round_body(r, carry2):
      drain_g(0)
      scatter(0)
      drain_grp()
      unpack_slot(0)
      fire(0)
      for t in range(1, NSLOT):
        drain_g(t)
        scatter(t)
        unpack_slot(t)
        fire(t)

      @pl.when(r < NROUND - 2)
      def _():
        load_grp(r + 2)
      return carry2

    lax.fori_loop(0, NROUND - 1, round_body, 0)
    for t in range(NSLOT):
      drain_g(t)
      scatter(t)
    plsc.subcore_barrier()

    out0 = (c * 2 + p) * N_PAD + base
    pltpu.sync_copy(acc_sh.at[pl.ds(base, ROWS_PER_TILE)],
                    sums_hbm.at[pl.ds(out0, ROWS_PER_TILE)])
    plsc.subcore_barrier()
    return carry

  lax.fori_loop(0, 2, phase_body, 0)


_sc_aggregate = pl.kernel(
    _sc_aggregate_body,
    out_type=jax.ShapeDtypeStruct((4 * N_PAD, HALF), jnp.float32),
    mesh=plsc.VectorSubcoreMesh(core_axis_name="c", subcore_axis_name="s"),
    compiler_params=pltpu.CompilerParams(use_tc_tiling_on_sc=False),
    scratch_types=[
        pltpu.VMEM((NSLOT * CHUNK,), jnp.int32),
        pltpu.VMEM((CHUNK, HALF), jnp.float32),
        [pltpu.VMEM((CHUNK,), jnp.int32) for _ in range(NSLOT)],
        [pltpu.VMEM((CHUNK,), jnp.int32) for _ in range(NSLOT)],
        [pltpu.VMEM((CHUNK, HALF), jnp.float32) for _ in range(NSLOT)],
        pltpu.VMEM_SHARED((N_PAD, HALF), jnp.float32),
        pltpu.VMEM_SHARED((N_PAD, HALF), jnp.float32),
        pltpu.SemaphoreType.DMA,
        [pltpu.SemaphoreType.DMA for _ in range(NSLOT)],
    ],
)


def _sc_counts_body(edges_hbm, cnts_hbm, pk0_v, pk1_v, idxg_v, idxs0_v,
                    idxs1_v, ones_v, zeros_v, cnt_sh, semp0, semp1):
  c = lax.axis_index("c")
  s = lax.axis_index("s")

  for j in range(128 // 16):
    zeros_v[pl.ds(j * 16, 16)] = jnp.zeros((16,), jnp.float32)
  for j in range(CHUNK // 16):
    ones_v[pl.ds(j * 16, 16)] = jnp.ones((16,), jnp.float32)

  base = s * ROWS_PER_TILE
  for k in range(ROWS_PER_TILE // 128):
    pltpu.sync_copy(zeros_v, cnt_sh.at[pl.ds(base + k * 128, 128)])
  plsc.subcore_barrier()

  ebase = s * (CPT * CHUNK)

  def load_pk(jj, pk_buf, sem):
    pltpu.async_copy(edges_hbm.at[pl.ds(ebase + jj * CHUNK, CHUNK)],
                     pk_buf, sem)

  def drain_pk(pk_buf, sem):
    pltpu.make_async_copy(edges_hbm.at[pl.ds(0, CHUNK)], pk_buf, sem).wait()

  def unpack(pk_v, idxs_v):
    for k in range(CHUNK // 16):
      p = pk_v[pl.ds(k * 16, 16)]
      lo = p & IDX_MASK
      hi = lax.shift_right_logical(p, IDX_BITS)
      idxs_v[pl.ds(k * 16, 16)] = jnp.where(c == 0, hi, lo)

  load_pk(0, pk0_v, semp0)

  def body(g, carry):
    j0 = 2 * g
    load_pk(j0 + 1, pk1_v, semp1)
    drain_pk(pk0_v, semp0)
    unpack(pk0_v, idxs0_v)

    @pl.when(g < CPT // 2 - 1)
    def _():
      load_pk(j0 + 2, pk0_v, semp0)

    pltpu.sync_copy(ones_v, cnt_sh.at[idxs0_v], add=True)
    drain_pk(pk1_v, semp1)
    unpack(pk1_v, idxs1_v)
    pltpu.sync_copy(ones_v, cnt_sh.at[idxs1_v], add=True)
    return carry

  lax.fori_loop(0, CPT // 2, body, 0)
  plsc.subcore_barrier()

  @pl.when(c == 0)
  def _():
    pltpu.sync_copy(cnt_sh.at[pl.ds(base, ROWS_PER_TILE)],
                    cnts_hbm.at[pl.ds(base, ROWS_PER_TILE)])

  @pl.when(c == 1)
  def _():
    pltpu.sync_copy(cnt_sh.at[pl.ds(base, ROWS_PER_TILE)],
                    cnts_hbm.at[pl.ds(N_PAD + base, ROWS_PER_TILE)])


_sc_counts = pl.kernel(
    _sc_counts_body,
    out_type=jax.ShapeDtypeStruct((2 * N_PAD,), jnp.float32),
    mesh=plsc.VectorSubcoreMesh(core_axis_name="c", subcore_axis_name="s"),
    scratch_types=[
        pltpu.VMEM((CHUNK,), jnp.int32),
        pltpu.VMEM((CHUNK,), jnp.int32),
        pltpu.VMEM((CHUNK,), jnp.int32),
        pltpu.VMEM((CHUNK,), jnp.int32),
        pltpu.VMEM((CHUNK,), jnp.int32),
        pltpu.VMEM((CHUNK,), jnp.float32),
        pltpu.VMEM((128,), jnp.float32),
        pltpu.VMEM_SHARED((N_PAD,), jnp.float32),
        pltpu.SemaphoreType.DMA,
        pltpu.SemaphoreType.DMA,
    ],
)


def _tc_layer_body(hlo_ref, hhi_ref, s00_ref, s01_ref, s10_ref, s11_ref,
                   c1_ref, c2_ref, w1t_ref, w2t_ref, wrt_ref, br_ref,
                   o_ref, *, blk, nb):
  r1 = 1.0 / jnp.maximum(c1_ref[...], 1.0)
  r2 = 1.0 / jnp.maximum(c2_ref[...], 1.0)
  h = jnp.concatenate([hlo_ref[...], hhi_ref[...]], axis=1)
  m1 = jnp.concatenate([s00_ref[...] * r1, s01_ref[...] * r1], axis=1)
  m2 = jnp.concatenate([s10_ref[...] * r2, s11_ref[...] * r2], axis=1)
  acc = jnp.dot(h, wrt_ref[...], preferred_element_type=jnp.float32)
  acc = acc + jnp.dot(m1, w1t_ref[...], preferred_element_type=jnp.float32)
  acc = acc + jnp.dot(m2, w2t_ref[...], preferred_element_type=jnp.float32)
  acc = acc + br_ref[...]
  acc = jnp.maximum(acc, 0.0)
  i = pl.program_id(0)
  rows = lax.broadcasted_iota(jnp.int32, acc.shape, 0) + (i % nb) * blk
  acc = jnp.where(rows < N, acc, 0.0)
  o_ref[...] = jnp.where(i < nb, acc[:, :HALF], acc[:, HALF:])


def _tc_layer(hst, sums_flat, cnts_flat, w1t, w2t, wrt, brl):
  blk = 1024
  nb = N_PAD // blk
  lo = lambda k: pl.BlockSpec((blk, HALF), lambda i, k=k: (i % nb + k * nb, 0))
  cspec = lambda k: pl.BlockSpec((blk, 1), lambda i, k=k: (i % nb + k * nb, 0))
  full_spec = lambda a, b: pl.BlockSpec((a, b), lambda i: (0, 0))
  return pl.pallas_call(
      functools.partial(_tc_layer_body, blk=blk, nb=nb),
      grid=(2 * nb,),
      in_specs=[
          lo(0), lo(1),
          lo(0), lo(1), lo(2), lo(3),
          cspec(0), cspec(1),
          full_spec(D, D), full_spec(D, D), full_spec(D, D), full_spec(1, D),
      ],
      out_specs=pl.BlockSpec((blk, HALF), lambda i: (i, 0)),
      out_shape=jax.ShapeDtypeStruct((2 * N_PAD, HALF), jnp.float32),
  )(hst, hst, sums_flat, sums_flat, sums_flat, sums_flat,
    cnts_flat, cnts_flat, w1t, w2t, wrt, brl)


def _tc_final_body(x_lo, x_hi, h1_lo, h1_hi, h2_lo, h2_hi, h3_lo, h3_hi,
                   wft_ref, bf_ref, o_ref):
  cat = jnp.concatenate(
      [x_lo[...], x_hi[...], h1_lo[...], h1_hi[...],
       h2_lo[...], h2_hi[...], h3_lo[...], h3_hi[...]], axis=1)
  o_ref[...] = jnp.dot(
      cat, wft_ref[...], preferred_element_type=jnp.float32) + bf_ref[...]


def _tc_final(stacked, wft, bf):
  blk = 1024
  nb = N_PAD // blk
  lo = pl.BlockSpec((blk, HALF), lambda i: (i, 0))
  hi = pl.BlockSpec((blk, HALF), lambda i, nb=nb: (i + nb, 0))
  args = []
  specs = []
  for a in stacked:
    args.extend([a, a])
    specs.extend([lo, hi])
  return pl.pallas_call(
      _tc_final_body,
      grid=(nb,),
      in_specs=specs + [
          pl.BlockSpec((L * D + D, D), lambda i: (0, 0)),
          pl.BlockSpec((1, D), lambda i: (0, 0)),
      ],
      out_specs=pl.BlockSpec((blk, D), lambda i: (i, 0)),
      out_shape=jax.ShapeDtypeStruct((N_PAD, D), jnp.float32),
  )(*args, wft, bf)


def kernel(x, edge_index, W1, W2, Wr, br, Wf, bf):
  x_pad = jnp.zeros((N_PAD, D), jnp.float32).at[:N].set(x)
  xst = jnp.concatenate([x_pad[:, :HALF], x_pad[:, HALF:]], axis=0)
  pad = jnp.full((E_PAD - E,), N, jnp.int32)
  src_p = jnp.concatenate([edge_index[0], pad])
  dst_p = jnp.concatenate([edge_index[1], pad])
  edges_p = src_p | (dst_p << IDX_BITS)
  w1t = jnp.transpose(W1, (0, 2, 1))
  w2t = jnp.transpose(W2, (0, 2, 1))
  wrt = jnp.transpose(Wr, (0, 2, 1))
  wft = jnp.transpose(Wf)
  hst = xst
  stacked = [xst]
  cnts_flat = _sc_counts(edges_p).reshape(2 * N_PAD, 1)
  for l in range(L):
    sums_flat = _sc_aggregate(hst, edges_p)
    hst = _tc_layer(hst, sums_flat, cnts_flat,
                    w1t[l], w2t[l], wrt[l], br[l].reshape(1, D))
    stacked.append(hst)
  out = _tc_final(stacked, wft, bf.reshape(1, D))
  return out[:N]
